# baseline (speedup 1.0000x reference)
import functools

import jax
import jax.numpy as jnp
from jax import lax
from jax.experimental import pallas as pl
from jax.experimental.pallas import tpu as pltpu
from jax.experimental.pallas import tpu_sc as plsc

BATCH = 16384
EMBED_DIM = 64
WIDE = 2 * EMBED_DIM
VOCAB = 1000000
COLS_PER_STEP = 2048
N_STEPS = 245
SPLIT = N_STEPS * COLS_PER_STEP
NUM_CORES = 2
NUM_SUBCORES = 16
NW = NUM_CORES * NUM_SUBCORES
B_PER_W = BATCH // NW
CHUNK = 128
N_CHUNK = B_PER_W // CHUNK
N_VEC = B_PER_W // 16


def _repack_body(t1_ref, t0_ref, g_ref):
    eye = jnp.eye(EMBED_DIM, dtype=jnp.float32)
    dn = (((0,), (0,)), ((), ()))
    a0 = lax.dot_general(t0_ref[...], eye, dn,
                         precision=lax.Precision.HIGHEST)
    a1 = lax.dot_general(t1_ref[...], eye, dn,
                         precision=lax.Precision.HIGHEST)
    g_ref[...] = jnp.concatenate([a0, a1], axis=1)


_repack = pl.pallas_call(
    _repack_body,
    grid=(N_STEPS,),
    in_specs=[
        pl.BlockSpec((EMBED_DIM, COLS_PER_STEP), lambda i: (0, i)),
        pl.BlockSpec(
            (EMBED_DIM, COLS_PER_STEP),
            lambda i: (0, jnp.minimum(i + N_STEPS, VOCAB // COLS_PER_STEP)),
        ),
    ],
    out_specs=pl.BlockSpec((COLS_PER_STEP, WIDE), lambda i: (i, 0)),
    out_shape=jax.ShapeDtypeStruct((SPLIT, WIDE), jnp.float32),
)

_MESH = plsc.VectorSubcoreMesh(core_axis_name="c", subcore_axis_name="s")


@functools.partial(
    pl.kernel,
    mesh=_MESH,
    out_type=jax.ShapeDtypeStruct((BATCH // 2, WIDE), jnp.float32),
    scratch_types=[
        pltpu.VMEM((N_CHUNK, CHUNK), jnp.int32),
        pltpu.VMEM((B_PER_W,), jnp.int32),
        pltpu.VMEM((B_PER_W, WIDE), jnp.float32),
        pltpu.VMEM((B_PER_W // 2, WIDE), jnp.float32),
        pltpu.SemaphoreType.DMA,
    ],
    compiler_params=pltpu.CompilerParams(
        use_tc_tiling_on_sc=True, needs_layout_passes=False
    ),
)
def _gather_kernel(idx2_hbm, half_hbm, table_hbm, out_hbm,
                   idx2_v, h_v, wide_v, rows_v, sem):
    wid = lax.axis_index("s") * NUM_CORES + lax.axis_index("c")
    base = wid * B_PER_W
    pltpu.sync_copy(idx2_hbm.at[pl.ds(wid * N_CHUNK, N_CHUNK)], idx2_v)
    pltpu.sync_copy(half_hbm.at[pl.ds(base, B_PER_W)], h_v)
    copies = [
        pltpu.async_copy(
            table_hbm.at[idx2_v.at[j]],
            wide_v.at[pl.ds(j * CHUNK, CHUNK)],
            sem,
        )
        for j in range(N_CHUNK)
    ]
    for c in copies:
        c.wait()
    lanes = lax.iota(jnp.int32, 16)

    def body(g, _):
        hv = h_v[pl.ds(g * 16, 16)]
        jv = g * 16 + lanes
        drow = jv >> 1
        dhalf = (jv & 1) * EMBED_DIM
        for c in range(EMBED_DIM):
            vals = plsc.load_gather(wide_v, [jv, hv + c])
            plsc.store_scatter(rows_v, [drow, dhalf + c], vals)
        return 0

    lax.fori_loop(0, N_VEC, body, 0)
    pltpu.sync_copy(rows_v, out_hbm.at[pl.ds(wid * (B_PER_W // 2), B_PER_W // 2)])


def kernel(x, table):
    table_t = table.T
    packed = _repack(table_t, table_t)
    in_hi = x >= SPLIT
    idx2 = jnp.where(in_hi, x - SPLIT, x).reshape(NW * N_CHUNK, CHUNK)
    half = jnp.where(in_hi, 0, EMBED_DIM).astype(jnp.int32)
    out_wide = _gather_kernel(idx2, half, packed)
    return out_wide.reshape(BATCH, EMBED_DIM)


# speedup vs baseline: 1.8326x; 1.8326x over previous
import functools

import jax
import jax.numpy as jnp
from jax import lax
from jax.experimental import pallas as pl
from jax.experimental.pallas import tpu as pltpu
from jax.experimental.pallas import tpu_sc as plsc

BATCH = 16384
EMBED_DIM = 64
WIDE = 2 * EMBED_DIM
VOCAB = 1000000
COLS_PER_STEP = 4096
N_STEPS = 123
SPLIT = N_STEPS * COLS_PER_STEP
NUM_CORES = 2
NUM_SUBCORES = 16
NW = NUM_CORES * NUM_SUBCORES
B_PER_W = BATCH // NW
CHUNK = 128
N_CHUNK = B_PER_W // CHUNK
N_VEC = B_PER_W // 16


def _repack_body(t1_ref, t0_ref, g_ref):
    g_ref[...] = jnp.concatenate([t0_ref[...].T, t1_ref[...].T], axis=1)


_repack = pl.pallas_call(
    _repack_body,
    grid=(N_STEPS,),
    in_specs=[
        pl.BlockSpec((EMBED_DIM, COLS_PER_STEP), lambda i: (0, i)),
        pl.BlockSpec(
            (EMBED_DIM, COLS_PER_STEP),
            lambda i: (0, jnp.minimum(i + N_STEPS, VOCAB // COLS_PER_STEP)),
        ),
    ],
    out_specs=pl.BlockSpec((COLS_PER_STEP, WIDE), lambda i: (i, 0)),
    out_shape=jax.ShapeDtypeStruct((SPLIT, WIDE), jnp.float32),
)

_MESH = plsc.VectorSubcoreMesh(core_axis_name="c", subcore_axis_name="s")


@functools.partial(
    pl.kernel,
    mesh=_MESH,
    out_type=jax.ShapeDtypeStruct((BATCH // 2, WIDE), jnp.float32),
    scratch_types=[
        pltpu.VMEM((N_CHUNK, CHUNK), jnp.int32),
        pltpu.VMEM((B_PER_W,), jnp.int32),
        pltpu.VMEM((B_PER_W, WIDE), jnp.float32),
        pltpu.VMEM((B_PER_W // 2, WIDE), jnp.float32),
        pltpu.SemaphoreType.DMA,
    ],
    compiler_params=pltpu.CompilerParams(
        use_tc_tiling_on_sc=True, needs_layout_passes=False
    ),
)
def _gather_kernel(idx2_hbm, half_hbm, table_hbm, out_hbm,
                   idx2_v, h_v, wide_v, rows_v, sem):
    wid = lax.axis_index("s") * NUM_CORES + lax.axis_index("c")
    base = wid * B_PER_W
    pltpu.sync_copy(idx2_hbm.at[pl.ds(wid * N_CHUNK, N_CHUNK)], idx2_v)
    pltpu.sync_copy(half_hbm.at[pl.ds(base, B_PER_W)], h_v)
    copies = [
        pltpu.async_copy(
            table_hbm.at[idx2_v.at[j]],
            wide_v.at[pl.ds(j * CHUNK, CHUNK)],
            sem,
        )
        for j in range(N_CHUNK)
    ]
    for c in copies:
        c.wait()
    lanes = lax.iota(jnp.int32, 16)

    def body(g, _):
        hv = h_v[pl.ds(g * 16, 16)]
        jv = g * 16 + lanes
        drow = jv >> 1
        dhalf = (jv & 1) * EMBED_DIM
        for c in range(EMBED_DIM):
            vals = plsc.load_gather(wide_v, [jv, hv + c])
            plsc.store_scatter(rows_v, [drow, dhalf + c], vals)
        return 0

    lax.fori_loop(0, N_VEC, body, 0)
    pltpu.sync_copy(rows_v, out_hbm.at[pl.ds(wid * (B_PER_W // 2), B_PER_W // 2)])


def kernel(x, table):
    table_t = table.T
    packed = _repack(table_t, table_t)
    in_hi = x >= SPLIT
    idx2 = jnp.where(in_hi, x - SPLIT, x).reshape(NW * N_CHUNK, CHUNK)
    half = jnp.where(in_hi, 0, EMBED_DIM).astype(jnp.int32)
    out_wide = _gather_kernel(idx2, half, packed)
    return out_wide.reshape(BATCH, EMBED_DIM)


# speedup vs baseline: 2.0272x; 1.1062x over previous
import functools

import jax
import jax.numpy as jnp
from jax import lax
from jax.experimental import pallas as pl
from jax.experimental.pallas import tpu as pltpu
from jax.experimental.pallas import tpu_sc as plsc

BATCH = 16384
EMBED_DIM = 64
WIDE = 2 * EMBED_DIM
VOCAB = 1000000
COLS_PER_STEP = 8192
N_STEPS = 62
SPLIT = N_STEPS * COLS_PER_STEP
NUM_CORES = 2
NUM_SUBCORES = 16
NW = NUM_CORES * NUM_SUBCORES
B_PER_W = BATCH // NW
CHUNK = 128
N_CHUNK = B_PER_W // CHUNK
N_VEC = B_PER_W // 16


def _repack_body(t1_ref, t0_ref, g_ref):
    g_ref[...] = jnp.concatenate([t0_ref[...].T, t1_ref[...].T], axis=1)


_repack = pl.pallas_call(
    _repack_body,
    grid=(N_STEPS,),
    in_specs=[
        pl.BlockSpec((EMBED_DIM, COLS_PER_STEP), lambda i: (0, i)),
        pl.BlockSpec(
            (EMBED_DIM, COLS_PER_STEP),
            lambda i: (0, jnp.minimum(i + N_STEPS, VOCAB // COLS_PER_STEP)),
        ),
    ],
    out_specs=pl.BlockSpec((COLS_PER_STEP, WIDE), lambda i: (i, 0)),
    out_shape=jax.ShapeDtypeStruct((SPLIT, WIDE), jnp.float32),
)

_MESH = plsc.VectorSubcoreMesh(core_axis_name="c", subcore_axis_name="s")


@functools.partial(
    pl.kernel,
    mesh=_MESH,
    out_type=jax.ShapeDtypeStruct((BATCH // 2, WIDE), jnp.float32),
    scratch_types=[
        pltpu.VMEM((N_CHUNK, CHUNK), jnp.int32),
        pltpu.VMEM((B_PER_W,), jnp.int32),
        pltpu.VMEM((B_PER_W, WIDE), jnp.float32),
        pltpu.VMEM((B_PER_W // 2, WIDE), jnp.float32),
        pltpu.SemaphoreType.DMA,
    ],
    compiler_params=pltpu.CompilerParams(
        use_tc_tiling_on_sc=True, needs_layout_passes=False
    ),
)
def _gather_kernel(idx2_hbm, half_hbm, table_hbm, out_hbm,
                   idx2_v, h_v, wide_v, rows_v, sem):
    wid = lax.axis_index("s") * NUM_CORES + lax.axis_index("c")
    base = wid * B_PER_W
    pltpu.sync_copy(idx2_hbm.at[pl.ds(wid * N_CHUNK, N_CHUNK)], idx2_v)
    pltpu.sync_copy(half_hbm.at[pl.ds(base, B_PER_W)], h_v)
    copies = [
        pltpu.async_copy(
            table_hbm.at[idx2_v.at[j]],
            wide_v.at[pl.ds(j * CHUNK, CHUNK)],
            sem,
        )
        for j in range(N_CHUNK)
    ]
    for c in copies:
        c.wait()
    lanes = lax.iota(jnp.int32, 16)

    def body(g, _):
        hv = h_v[pl.ds(g * 16, 16)]
        jv = g * 16 + lanes
        drow = jv >> 1
        dhalf = (jv & 1) * EMBED_DIM
        for c in range(EMBED_DIM):
            vals = plsc.load_gather(wide_v, [jv, hv + c])
            plsc.store_scatter(rows_v, [drow, dhalf + c], vals)
        return 0

    lax.fori_loop(0, N_VEC, body, 0)
    pltpu.sync_copy(rows_v, out_hbm.at[pl.ds(wid * (B_PER_W // 2), B_PER_W // 2)])


def kernel(x, table):
    table_t = table.T
    packed = _repack(table_t, table_t)
    in_hi = x >= SPLIT
    idx2 = jnp.where(in_hi, x - SPLIT, x).reshape(NW * N_CHUNK, CHUNK)
    half = jnp.where(in_hi, 0, EMBED_DIM).astype(jnp.int32)
    out_wide = _gather_kernel(idx2, half, packed)
    return out_wide.reshape(BATCH, EMBED_DIM)


# speedup vs baseline: 2.1289x; 1.0502x over previous
import functools

import jax
import jax.numpy as jnp
from jax import lax
from jax.experimental import pallas as pl
from jax.experimental.pallas import tpu as pltpu
from jax.experimental.pallas import tpu_sc as plsc

BATCH = 16384
EMBED_DIM = 64
WIDE = 2 * EMBED_DIM
VOCAB = 1000000
COLS_PER_STEP = 16384
N_STEPS = 31
SPLIT = N_STEPS * COLS_PER_STEP
NUM_CORES = 2
NUM_SUBCORES = 16
NW = NUM_CORES * NUM_SUBCORES
B_PER_W = BATCH // NW
CHUNK = 128
N_CHUNK = B_PER_W // CHUNK
N_VEC = B_PER_W // 16


def _repack_body(t1_ref, t0_ref, g_ref):
    g_ref[...] = jnp.concatenate([t0_ref[...].T, t1_ref[...].T], axis=1)


_repack = pl.pallas_call(
    _repack_body,
    grid=(N_STEPS,),
    in_specs=[
        pl.BlockSpec((EMBED_DIM, COLS_PER_STEP), lambda i: (0, i)),
        pl.BlockSpec(
            (EMBED_DIM, COLS_PER_STEP),
            lambda i: (0, jnp.minimum(i + N_STEPS, VOCAB // COLS_PER_STEP)),
        ),
    ],
    out_specs=pl.BlockSpec((COLS_PER_STEP, WIDE), lambda i: (i, 0)),
    out_shape=jax.ShapeDtypeStruct((SPLIT, WIDE), jnp.float32),
)

_MESH = plsc.VectorSubcoreMesh(core_axis_name="c", subcore_axis_name="s")


@functools.partial(
    pl.kernel,
    mesh=_MESH,
    out_type=jax.ShapeDtypeStruct((BATCH // 2, WIDE), jnp.float32),
    scratch_types=[
        pltpu.VMEM((N_CHUNK, CHUNK), jnp.int32),
        pltpu.VMEM((B_PER_W,), jnp.int32),
        pltpu.VMEM((B_PER_W, WIDE), jnp.float32),
        pltpu.VMEM((B_PER_W // 2, WIDE), jnp.float32),
        pltpu.SemaphoreType.DMA,
    ],
    compiler_params=pltpu.CompilerParams(
        use_tc_tiling_on_sc=True, needs_layout_passes=False
    ),
)
def _gather_kernel(idx2_hbm, half_hbm, table_hbm, out_hbm,
                   idx2_v, h_v, wide_v, rows_v, sem):
    wid = lax.axis_index("s") * NUM_CORES + lax.axis_index("c")
    base = wid * B_PER_W
    pltpu.sync_copy(idx2_hbm.at[pl.ds(wid * N_CHUNK, N_CHUNK)], idx2_v)
    pltpu.sync_copy(half_hbm.at[pl.ds(base, B_PER_W)], h_v)
    copies = [
        pltpu.async_copy(
            table_hbm.at[idx2_v.at[j]],
            wide_v.at[pl.ds(j * CHUNK, CHUNK)],
            sem,
        )
        for j in range(N_CHUNK)
    ]
    for c in copies:
        c.wait()
    lanes = lax.iota(jnp.int32, 16)

    def body(g, _):
        hv = h_v[pl.ds(g * 16, 16)]
        jv = g * 16 + lanes
        drow = jv >> 1
        dhalf = (jv & 1) * EMBED_DIM
        for c in range(EMBED_DIM):
            vals = plsc.load_gather(wide_v, [jv, hv + c])
            plsc.store_scatter(rows_v, [drow, dhalf + c], vals)
        return 0

    lax.fori_loop(0, N_VEC, body, 0)
    pltpu.sync_copy(rows_v, out_hbm.at[pl.ds(wid * (B_PER_W // 2), B_PER_W // 2)])


def kernel(x, table):
    table_t = table.T
    packed = _repack(table_t, table_t)
    in_hi = x >= SPLIT
    idx2 = jnp.where(in_hi, x - SPLIT, x).reshape(NW * N_CHUNK, CHUNK)
    half = jnp.where(in_hi, 0, EMBED_DIM).astype(jnp.int32)
    out_wide = _gather_kernel(idx2, half, packed)
    return out_wide.reshape(BATCH, EMBED_DIM)
